# baseline (speedup 1.0000x reference)
import jax
import jax.numpy as jnp
from jax import lax
from jax.experimental import pallas as pl
from jax.experimental.pallas import tpu as pltpu
from jax.experimental.pallas import tpu_sc as plsc

VOCAB = 1000
D_MODEL = 64
SEQ = 50
LANES = 16
NUM_CORES = 2
NUM_SUBCORES = 16
NUM_WORKERS = NUM_CORES * NUM_SUBCORES
ROWS_PER_W = 1024 // NUM_WORKERS
NG = 4
NS = 4


def _squeeze(slots, gb, staging, sb):
    def body(r, carry):
        for c in range(D_MODEL // LANES):
            staging[sb, r, pl.ds(c * LANES, LANES)] = (
                slots[gb, r, pl.ds(c * LANES, LANES)]
            )
        return carry
    lax.fori_loop(0, SEQ, body, 0, unroll=2)


def _emb_body(idx_hbm, table_hbm, out_hbm, idx_v, slots, staging, gsem, osem):
    wid = lax.axis_index("s") * NUM_CORES + lax.axis_index("c")
    base = wid * ROWS_PER_W
    pltpu.sync_copy(idx_hbm.at[pl.ds(base, ROWS_PER_W)], idx_v)

    def gather(j):
        return pltpu.async_copy(table_hbm.at[idx_v.at[j]], slots.at[j % NG], gsem)

    def put(j):
        return pltpu.async_copy(
            staging.at[j % NS], out_hbm.at[base + j], osem
        )

    gets = [gather(j) for j in range(NG)]
    puts = []
    for j in range(ROWS_PER_W):
        gets[j].wait()
        if j >= NS:
            puts[j - NS].wait()
        _squeeze(slots, j % NG, staging, j % NS)
        puts.append(put(j))
        nj = j + NG
        if nj < ROWS_PER_W:
            gets.append(gather(nj))
    for j in range(ROWS_PER_W - NS, ROWS_PER_W):
        puts[j].wait()


@jax.jit
def kernel(token_ids, w):
    nxt = jnp.concatenate([w[1:], jnp.zeros((1, D_MODEL), w.dtype)], axis=0)
    table2 = jnp.concatenate([w, nxt], axis=1)
    grab = pl.kernel(
        _emb_body,
        out_type=jax.ShapeDtypeStruct((1024, SEQ, D_MODEL), jnp.float32),
        mesh=plsc.VectorSubcoreMesh(
            core_axis_name="c",
            subcore_axis_name="s",
            num_cores=NUM_CORES,
            num_subcores=NUM_SUBCORES,
        ),
        scratch_types=[
            pltpu.VMEM((ROWS_PER_W, SEQ), jnp.int32),
            pltpu.VMEM((NG, SEQ, 2 * D_MODEL), jnp.float32),
            pltpu.VMEM((NS, SEQ, D_MODEL), jnp.float32),
            pltpu.SemaphoreType.DMA,
            pltpu.SemaphoreType.DMA,
        ],
        compiler_params=pltpu.CompilerParams(use_tc_tiling_on_sc=True),
    )
    return grab(token_ids, table2)


# speedup vs baseline: 1.0002x; 1.0002x over previous
import jax
import jax.numpy as jnp
from jax import lax
from jax.experimental import pallas as pl
from jax.experimental.pallas import tpu as pltpu
from jax.experimental.pallas import tpu_sc as plsc

VOCAB = 1000
D_MODEL = 64
SEQ = 50
LANES = 16
NUM_CORES = 2
NUM_SUBCORES = 16
NUM_WORKERS = NUM_CORES * NUM_SUBCORES
ROWS_PER_W = 1024 // NUM_WORKERS
NG = 4
NS = 4


def _squeeze(slots, gb, staging, sb):
    def body(r, carry):
        for c in range(D_MODEL // LANES):
            staging[sb, r, pl.ds(c * LANES, LANES)] = (
                slots[gb, r, pl.ds(c * LANES, LANES)]
            )
        return carry
    lax.fori_loop(0, SEQ, body, 0, unroll=2)


def _emb_body(idx_hbm, table_hbm, out_hbm, idx_v, slots, staging, gsem, osem):
    wid = lax.axis_index("s") * NUM_CORES + lax.axis_index("c")
    base = wid * ROWS_PER_W
    pltpu.sync_copy(idx_hbm.at[pl.ds(base, ROWS_PER_W)], idx_v)

    def gather(j):
        return pltpu.async_copy(table_hbm.at[idx_v.at[j]], slots.at[j % NG], gsem)

    def put(j):
        return pltpu.async_copy(
            staging.at[j % NS], out_hbm.at[base + j], osem
        )

    gets = [gather(j) for j in range(NG)]
    puts = []
    for j in range(ROWS_PER_W):
        gets[j].wait()
        if j >= NS:
            puts[j - NS].wait()
        _squeeze(slots, j % NG, staging, j % NS)
        puts.append(put(j))
        nj = j + NG
        if nj < ROWS_PER_W:
            gets.append(gather(nj))
    for j in range(ROWS_PER_W - NS, ROWS_PER_W):
        puts[j].wait()


def kernel(token_ids, w):
    nxt = jnp.concatenate([w[1:], jnp.zeros((1, D_MODEL), w.dtype)], axis=0)
    table2 = jnp.concatenate([w, nxt], axis=1)
    grab = pl.kernel(
        _emb_body,
        out_type=jax.ShapeDtypeStruct((1024, SEQ, D_MODEL), jnp.float32),
        mesh=plsc.VectorSubcoreMesh(
            core_axis_name="c",
            subcore_axis_name="s",
            num_cores=NUM_CORES,
            num_subcores=NUM_SUBCORES,
        ),
        scratch_types=[
            pltpu.VMEM((ROWS_PER_W, SEQ), jnp.int32),
            pltpu.VMEM((NG, SEQ, 2 * D_MODEL), jnp.float32),
            pltpu.VMEM((NS, SEQ, D_MODEL), jnp.float32),
            pltpu.SemaphoreType.DMA,
            pltpu.SemaphoreType.DMA,
        ],
        compiler_params=pltpu.CompilerParams(use_tc_tiling_on_sc=True),
    )
    return grab(token_ids, table2)
